# baseline (speedup 1.0000x reference)
import functools

import jax
import jax.numpy as jnp
from jax import lax
from jax.experimental import pallas as pl
from jax.experimental.pallas import tpu as pltpu
from jax.experimental.pallas import tpu_sc as plsc


def _widen_table(table, V, D, BK):
    def k1(tref, oref):
        x = tref[...]
        oref[...] = jnp.concatenate([x, x], axis=1)

    return pl.pallas_call(
        k1,
        grid=(V // BK,),
        in_specs=[pl.BlockSpec((BK, D), lambda i: (i, 0))],
        out_specs=pl.BlockSpec((BK, 2 * D), lambda i: (i, 0)),
        out_shape=jax.ShapeDtypeStruct((V, 2 * D), jnp.float32),
    )(table)


def _make_gather(NF, NW, NC, CR, NB):
    r_per_w = NF // NW
    n_chunks = r_per_w // CR
    LOOK = NB // 2
    mesh = plsc.VectorSubcoreMesh(core_axis_name="c", subcore_axis_name="s")

    scratch = [pltpu.VMEM((r_per_w,), jnp.int32)]
    scratch += [pltpu.VMEM((CR, 128), jnp.float32) for _ in range(NB)]
    scratch += [pltpu.SemaphoreType.DMA for _ in range(NB)]
    scratch += [pltpu.SemaphoreType.DMA for _ in range(NB)]

    @functools.partial(
        pl.kernel,
        mesh=mesh,
        compiler_params=pltpu.CompilerParams(use_tc_tiling_on_sc=False),
        out_type=jax.ShapeDtypeStruct((NF, 128), jnp.float32),
        scratch_types=scratch,
    )
    def k(idx_hbm, t128_hbm, out_hbm, idx_v, *rest):
        bufs = rest[:NB]
        sg = rest[NB : 2 * NB]
        sw = rest[2 * NB : 3 * NB]
        wid = lax.axis_index("s") * NC + lax.axis_index("c")
        base = wid * r_per_w
        pltpu.sync_copy(idx_hbm.at[pl.ds(wid * r_per_w, r_per_w)], idx_v)

        def gather(c, n):
            pltpu.async_copy(
                t128_hbm.at[idx_v.at[pl.ds(c * CR, CR)]], bufs[n], sg[n]
            )

        def gather_wait(n):
            pltpu.make_async_copy(
                t128_hbm.at[idx_v.at[pl.ds(0, CR)]], bufs[n], sg[n]
            ).wait()

        def write(c, n):
            pltpu.async_copy(
                bufs[n], out_hbm.at[pl.ds(base + c * CR, CR)], sw[n]
            )

        def write_wait(n):
            pltpu.make_async_copy(
                bufs[n], out_hbm.at[pl.ds(base, CR)], sw[n]
            ).wait()

        for n in range(LOOK):
            gather(n, n)

        T = n_chunks // NB

        def body(t, carry):
            for n in range(NB):
                j = t * NB + n
                m = (n + LOOK) % NB
                gather_wait(n)
                write(j, n)

                @pl.when(j >= LOOK)
                def _():
                    write_wait(m)

                @pl.when(j + LOOK < n_chunks)
                def _():
                    gather(j + LOOK, m)

            return carry

        lax.fori_loop(0, T, body, 0)
        for j in range(n_chunks - LOOK, n_chunks):
            write_wait(j % NB)

    return k


def kernel(input_batch, table):
    B, H = input_batch.shape
    V, D = table.shape
    HP = 56

    info = plsc.get_sparse_core_info()
    NC, NS = info.num_cores, info.num_subcores
    NW = NC * NS
    NB = 4
    CB = 4
    CR = CB * HP

    t128 = _widen_table(table, V, D, 8000)
    idxp = jnp.pad(input_batch.astype(jnp.int32), ((0, 0), (0, HP - H)))
    idxf = idxp.reshape(B * HP)
    out = _make_gather(B * HP, NW, NC, CR, NB)(idxf, t128)
    return out.reshape(B, HP, 2 * D)[:, :H, :D]
